# baseline (speedup 1.0000x reference)
import jax
import jax.numpy as jnp
from jax import lax
from jax.experimental import pallas as pl
from jax.experimental.pallas import tpu as pltpu
from jax.experimental.pallas import tpu_sc as plsc

_B = 16384
_D = 64
_NF = 14
_OUT_W = (_NF + 1) * _D
_NW = 32
_CB = 128
_NITEM = (_NF + 1) * (_B // _CB) // _NW
_L = 16

_GATHER_DNUMS = lax.GatherDimensionNumbers(
    offset_dims=(), collapsed_slice_dims=(0,), start_index_map=(0,))


def _lane_bcast(v, lane):
    if isinstance(lane, int):
        idx = jnp.full((_L, 1), lane, dtype=jnp.int32)
    else:
        idx = jnp.broadcast_to(lane, (_L,)).astype(jnp.int32)[:, None]
    return lax.gather(v, idx, _GATHER_DNUMS, (1,),
                      mode=lax.GatherScatterMode.PROMISE_IN_BOUNDS)


def _tower_body(sid_hbm, table_hbm, feats_hbm, w_hbm, b_hbm, out_hbm,
                idx_v, w_v, b_v, gbuf, fbuf, blk_v, psem, gsem, fsem, osem):
    c_ax = lax.axis_index("c")
    s_ax = lax.axis_index("s")
    w = s_ax * 2 + c_ax

    def chunk_col(i):
        return w + 32 * lax.rem(i, 4)

    pro = []
    for j in range(4):
        cj = w + 32 * j
        pro.append(pltpu.async_copy(sid_hbm.at[pl.ds(cj * _CB, _CB)],
                                    idx_v.at[j], psem))
    pro.append(pltpu.async_copy(w_hbm, w_v, psem))
    pro.append(pltpu.async_copy(b_hbm, b_v, psem))
    for cp in pro:
        cp.wait()

    def fire_input(j, jp):
        @pl.when(j < 4)
        def _():
            pltpu.async_copy(table_hbm.at[idx_v.at[j]], gbuf.at[jp],
                             gsem.at[jp])

        @pl.when(j >= 4)
        def _():
            f = lax.div(j, 4) - 1
            pltpu.async_copy(
                feats_hbm.at[f, pl.ds(chunk_col(j) * _CB, _CB)],
                fbuf.at[jp], fsem.at[jp])

    def out_slice(i):
        return out_hbm.at[pl.ds(lax.div(i, 4) * _D, _D),
                          pl.ds(chunk_col(i) * _CB, _CB)]

    fire_input(0, 0)

    def item(i, carry):
        p = lax.rem(i, 2)
        blk = blk_v.at[p]

        @pl.when(i + 1 < _NITEM)
        def _():
            fire_input(i + 1, 1 - p)

        @pl.when(i >= 2)
        def _():
            pltpu.make_async_copy(blk, out_slice(i - 2), osem.at[p]).wait()

        @pl.when(i < 4)
        def _():
            pltpu.make_async_copy(table_hbm.at[idx_v.at[i]], gbuf.at[p],
                                  gsem.at[p]).wait()
            iota = lax.iota(jnp.int32, _L)

            def row(r, carry2):
                cols = jnp.broadcast_to(r, (_L,)).astype(jnp.int32)
                for q in range(_D // _L):
                    val = gbuf[p, r, pl.ds(q * _L, _L)]
                    plsc.store_scatter(blk, [iota + (q * _L), cols], val)
                return carry2

            lax.fori_loop(0, _CB, row, 0)

        @pl.when(i >= 4)
        def _():
            f = lax.div(i, 4) - 1
            pltpu.make_async_copy(
                feats_hbm.at[f, pl.ds(chunk_col(i) * _CB, _CB)],
                fbuf.at[p], fsem.at[p]).wait()
            fvs = [fbuf[p, pl.ds(rc * _L, _L)] for rc in range(_CB // _L)]

            def qloop(q, carry2):
                wv = w_v[f, pl.ds(q * _L, _L)]
                bv = b_v[f, pl.ds(q * _L, _L)]
                for d2 in range(_L):
                    wbc = _lane_bcast(wv, d2)
                    bbc = _lane_bcast(bv, d2)
                    drow = q * _L + d2
                    for rc in range(_CB // _L):
                        blk[drow, pl.ds(rc * _L, _L)] = wbc * fvs[rc] + bbc
                return carry2

            lax.fori_loop(0, _D // _L, qloop, 0)

        pltpu.async_copy(blk, out_slice(i), osem.at[p])
        return carry

    lax.fori_loop(0, _NITEM, item, 0)

    for j in (_NITEM - 2, _NITEM - 1):
        pltpu.make_async_copy(blk_v.at[j % 2], out_slice(j),
                              osem.at[j % 2]).wait()


@jax.jit
def _tower(sid, table, feats, wmat, bmat):
    kern = pl.kernel(
        _tower_body,
        out_type=jax.ShapeDtypeStruct((_OUT_W, _B), jnp.float32),
        mesh=plsc.VectorSubcoreMesh(core_axis_name="c", subcore_axis_name="s"),
        compiler_params=pltpu.CompilerParams(needs_layout_passes=False,
                                             use_tc_tiling_on_sc=False),
        scratch_types=[
            pltpu.VMEM((4, _CB), jnp.int32),
            pltpu.VMEM((_NF, _D), jnp.float32),
            pltpu.VMEM((_NF, _D), jnp.float32),
            pltpu.VMEM((2, _CB, _D), jnp.float32),
            pltpu.VMEM((2, _CB), jnp.float32),
            pltpu.VMEM((2, _D, _CB), jnp.float32),
            pltpu.SemaphoreType.DMA,
            pltpu.SemaphoreType.DMA((2,)),
            pltpu.SemaphoreType.DMA((2,)),
            pltpu.SemaphoreType.DMA((2,)),
        ],
    )
    return kern(sid, table, feats, wmat, bmat)


def kernel(student_id, table,
           feat_age, feat_gender, feat_ethnicity, feat_location, feat_gpa,
           feat_test_scores, feat_courses, feat_major, feat_attendance,
           feat_participation, feat_feedback, feat_study_habits,
           feat_social_activity, feat_stress_level,
           W_age, W_gender, W_ethnicity, W_location, W_gpa,
           W_test_scores, W_courses, W_major, W_attendance,
           W_participation, W_feedback, W_study_habits,
           W_social_activity, W_stress_level,
           b_age, b_gender, b_ethnicity, b_location, b_gpa,
           b_test_scores, b_courses, b_major, b_attendance,
           b_participation, b_feedback, b_study_habits,
           b_social_activity, b_stress_level):
    feats = jnp.stack([
        feat_age, feat_gender, feat_ethnicity, feat_location, feat_gpa,
        feat_test_scores, feat_courses, feat_major, feat_attendance,
        feat_participation, feat_feedback, feat_study_habits,
        feat_social_activity, feat_stress_level])
    wmat = jnp.concatenate([
        W_age, W_gender, W_ethnicity, W_location, W_gpa,
        W_test_scores, W_courses, W_major, W_attendance,
        W_participation, W_feedback, W_study_habits,
        W_social_activity, W_stress_level], axis=0)
    bmat = jnp.stack([
        b_age, b_gender, b_ethnicity, b_location, b_gpa,
        b_test_scores, b_courses, b_major, b_attendance,
        b_participation, b_feedback, b_study_habits,
        b_social_activity, b_stress_level])
    out2 = _tower(student_id, table, feats, wmat, bmat)
    return out2.T


# speedup vs baseline: 1.7474x; 1.7474x over previous
import jax
import jax.numpy as jnp
from jax import lax
from jax.experimental import pallas as pl
from jax.experimental.pallas import tpu as pltpu
from jax.experimental.pallas import tpu_sc as plsc

_B = 16384
_D = 64
_NF = 14
_OUT_W = (_NF + 1) * _D
_NW = 32
_CB = 128
_NITEM = (_NF + 1) * (_B // _CB) // _NW
_L = 16

_GATHER_DNUMS = lax.GatherDimensionNumbers(
    offset_dims=(), collapsed_slice_dims=(0,), start_index_map=(0,))


def _lane_bcast(v, lane):
    if isinstance(lane, int):
        idx = jnp.full((_L, 1), lane, dtype=jnp.int32)
    else:
        idx = jnp.broadcast_to(lane, (_L,)).astype(jnp.int32)[:, None]
    return lax.gather(v, idx, _GATHER_DNUMS, (1,),
                      mode=lax.GatherScatterMode.PROMISE_IN_BOUNDS)


def _tower_body(sid_hbm, table_hbm, feats_hbm, w_hbm, b_hbm, out_hbm,
                idx_v, w_v, b_v, gbuf, fbuf, blk_v, psem, gsem, fsem, osem):
    c_ax = lax.axis_index("c")
    s_ax = lax.axis_index("s")
    w = s_ax * 2 + c_ax

    def chunk_col(i):
        return w + 32 * lax.rem(i, 4)

    pro = []
    for j in range(4):
        cj = w + 32 * j
        pro.append(pltpu.async_copy(sid_hbm.at[pl.ds(cj * _CB, _CB)],
                                    idx_v.at[j], psem))
    pro.append(pltpu.async_copy(w_hbm, w_v, psem))
    pro.append(pltpu.async_copy(b_hbm, b_v, psem))
    for cp in pro:
        cp.wait()

    def fire_input(j, jp):
        @pl.when(j < 4)
        def _():
            pltpu.async_copy(table_hbm.at[idx_v.at[j]], gbuf.at[jp],
                             gsem.at[jp])

        @pl.when(j >= 4)
        def _():
            f = lax.div(j, 4) - 1
            pltpu.async_copy(
                feats_hbm.at[f, pl.ds(chunk_col(j) * _CB, _CB)],
                fbuf.at[jp], fsem.at[jp])

    def out_slice(i):
        return out_hbm.at[pl.ds(lax.div(i, 4) * _D, _D),
                          pl.ds(chunk_col(i) * _CB, _CB)]

    fire_input(0, 0)

    def item(i, carry):
        p = lax.rem(i, 2)
        blk = blk_v.at[p]

        @pl.when(i + 1 < _NITEM)
        def _():
            fire_input(i + 1, 1 - p)

        @pl.when(i >= 2)
        def _():
            pltpu.make_async_copy(blk, out_slice(i - 2), osem.at[p]).wait()

        @pl.when(i < 4)
        def _():
            pltpu.make_async_copy(table_hbm.at[idx_v.at[i]], gbuf.at[p],
                                  gsem.at[p]).wait()
            iota = lax.iota(jnp.int32, _L)

            def row(r, carry2):
                cols = jnp.broadcast_to(r, (_L,)).astype(jnp.int32)
                for q in range(_D // _L):
                    val = gbuf[p, r, pl.ds(q * _L, _L)]
                    plsc.store_scatter(blk, [iota + (q * _L), cols], val)
                return carry2

            lax.fori_loop(0, _CB, row, 0)

        @pl.when(i >= 4)
        def _():
            f = lax.div(i, 4) - 1
            pltpu.make_async_copy(
                feats_hbm.at[f, pl.ds(chunk_col(i) * _CB, _CB)],
                fbuf.at[p], fsem.at[p]).wait()
            fvs = [fbuf[p, pl.ds(rc * _L, _L)] for rc in range(_CB // _L)]

            def qloop(q, carry2):
                wv = w_v[f, pl.ds(q * _L, _L)]
                bv = b_v[f, pl.ds(q * _L, _L)]
                for d2 in range(_L):
                    wbc = _lane_bcast(wv, d2)
                    bbc = _lane_bcast(bv, d2)
                    drow = q * _L + d2
                    for rc in range(_CB // _L):
                        blk[drow, pl.ds(rc * _L, _L)] = wbc * fvs[rc] + bbc
                return carry2

            lax.fori_loop(0, _D // _L, qloop, 0)

        pltpu.async_copy(blk, out_slice(i), osem.at[p])
        return carry

    lax.fori_loop(0, _NITEM, item, 0)

    for j in (_NITEM - 2, _NITEM - 1):
        pltpu.make_async_copy(blk_v.at[j % 2], out_slice(j),
                              osem.at[j % 2]).wait()


@jax.jit
def _tower(sid, table128, feats, wmat, bmat):
    kern = pl.kernel(
        _tower_body,
        out_type=jax.ShapeDtypeStruct((_OUT_W, _B), jnp.float32),
        mesh=plsc.VectorSubcoreMesh(core_axis_name="c", subcore_axis_name="s"),
        compiler_params=pltpu.CompilerParams(needs_layout_passes=False),
        scratch_types=[
            pltpu.VMEM((4, _CB), jnp.int32),
            pltpu.VMEM((_NF, _D), jnp.float32),
            pltpu.VMEM((_NF, _D), jnp.float32),
            pltpu.VMEM((2, _CB, 2 * _D), jnp.float32),
            pltpu.VMEM((2, _CB), jnp.float32),
            pltpu.VMEM((2, _D, _CB), jnp.float32),
            pltpu.SemaphoreType.DMA,
            pltpu.SemaphoreType.DMA((2,)),
            pltpu.SemaphoreType.DMA((2,)),
            pltpu.SemaphoreType.DMA((2,)),
        ],
    )
    return kern(sid, table128, feats, wmat, bmat)


def kernel(student_id, table,
           feat_age, feat_gender, feat_ethnicity, feat_location, feat_gpa,
           feat_test_scores, feat_courses, feat_major, feat_attendance,
           feat_participation, feat_feedback, feat_study_habits,
           feat_social_activity, feat_stress_level,
           W_age, W_gender, W_ethnicity, W_location, W_gpa,
           W_test_scores, W_courses, W_major, W_attendance,
           W_participation, W_feedback, W_study_habits,
           W_social_activity, W_stress_level,
           b_age, b_gender, b_ethnicity, b_location, b_gpa,
           b_test_scores, b_courses, b_major, b_attendance,
           b_participation, b_feedback, b_study_habits,
           b_social_activity, b_stress_level):
    feats = jnp.stack([
        feat_age, feat_gender, feat_ethnicity, feat_location, feat_gpa,
        feat_test_scores, feat_courses, feat_major, feat_attendance,
        feat_participation, feat_feedback, feat_study_habits,
        feat_social_activity, feat_stress_level])
    wmat = jnp.concatenate([
        W_age, W_gender, W_ethnicity, W_location, W_gpa,
        W_test_scores, W_courses, W_major, W_attendance,
        W_participation, W_feedback, W_study_habits,
        W_social_activity, W_stress_level], axis=0)
    bmat = jnp.stack([
        b_age, b_gender, b_ethnicity, b_location, b_gpa,
        b_test_scores, b_courses, b_major, b_attendance,
        b_participation, b_feedback, b_study_habits,
        b_social_activity, b_stress_level])
    table128 = jnp.concatenate(
        [table, jnp.zeros((table.shape[0], _D), table.dtype)], axis=1)
    out2 = _tower(student_id, table128, feats, wmat, bmat)
    return out2.T
